# baseline (speedup 1.0000x reference)
import functools

import jax
import jax.numpy as jnp
from jax import lax
from jax.experimental import pallas as pl
from jax.experimental.pallas import tpu as pltpu
from jax.experimental.pallas import tpu_sc as plsc

_NC = 2
_NS = 16
_NW = _NC * _NS
_LANES = 16
_NACT = 25
_BPW = 256
_GROUPS = _BPW // _LANES
_SEQLEN = 50


def _sc_gather_body(txt_hbm, tgt_hbm, outv_hbm, outm_hbm,
                    tgt_v, prev_v, ext_v, buf_a, buf_b, vred_v, mred_v,
                    sem_a, sem_b):
    wid = lax.axis_index("s") * _NC + lax.axis_index("c")
    lane = lax.iota(jnp.int32, _LANES)

    @pl.when(wid < _NACT)
    def _active():
        pltpu.sync_copy(tgt_hbm.at[wid], tgt_v)
        pltpu.sync_copy(
            tgt_hbm.at[jnp.maximum(wid - 1, 0), pl.ds(_BPW - _LANES, _LANES)],
            prev_v)
        ext_v[pl.ds(0, _LANES)] = prev_v[...]
        for g in range(_GROUPS):
            ext_v[pl.ds(_LANES + g * _LANES, _LANES)] = (
                tgt_v[pl.ds(g * _LANES, _LANES)])
        row0 = wid * _BPW
        bufs = (buf_a, buf_b)
        sems = (sem_a, sem_b)

        tvecs = [tgt_v[pl.ds(g * _LANES, _LANES)] for g in range(_GROUPS)]
        tscs = [[tvecs[g][l] for l in range(_LANES)] for g in range(_GROUPS)]
        mvecs = []
        for g in range(_GROUPS):
            prev = ext_v[pl.ds(_LANES + g * _LANES - 1, _LANES)]
            kvec = row0 + g * _LANES + lane
            first = (kvec % _SEQLEN) == 0
            mvecs.append(
                jnp.where(jnp.logical_or(first, prev > 0), 1.0, 0.0))

        def fire(g):
            buf, sem = bufs[g & 1], sems[g & 1]
            cb = pl.multiple_of((row0 + g * _LANES) & ~127, 128)
            return [pltpu.async_copy(
                txt_hbm.at[pl.ds(pl.multiple_of(tscs[g][l] & ~7, 8), 8),
                           pl.ds(cb, 128)],
                buf.at[l], sem) for l in range(_LANES)]

        accv = jnp.zeros((_LANES,), jnp.float32)
        accm = jnp.zeros((_LANES,), jnp.float32)
        cps = {0: fire(0)}
        for g in range(_GROUPS):
            if g + 1 < _GROUPS:
                cps[g + 1] = fire(g + 1)
            buf = bufs[g & 1]
            for l in range(_LANES):
                cps[g][l].wait()
                jt = tscs[g][l] & 7
                sel = jnp.where(lane == l, mvecs[g][l], 0.0)
                for s in range(8):
                    chunk = buf[l, s, pl.ds((g & 7) * _LANES, _LANES)]
                    accv = accv + chunk * jnp.where(jt == s, sel, 0.0)
            accm = accm + mvecs[g]
        vred_v[...] = accv
        mred_v[...] = accm
        pltpu.sync_copy(vred_v, outv_hbm.at[wid])
        pltpu.sync_copy(mred_v, outm_hbm.at[wid])


@functools.lru_cache(maxsize=1)
def _sc_gather():
    return pl.kernel(
        _sc_gather_body,
        out_type=[jax.ShapeDtypeStruct((_NACT, _LANES), jnp.float32),
                  jax.ShapeDtypeStruct((_NACT, _LANES), jnp.float32)],
        mesh=plsc.VectorSubcoreMesh(core_axis_name="c", subcore_axis_name="s",
                                    num_cores=_NC),
        scratch_types=[
            pltpu.VMEM((_BPW,), jnp.int32),
            pltpu.VMEM((_LANES,), jnp.int32),
            pltpu.VMEM((_LANES + _BPW,), jnp.int32),
            pltpu.VMEM((_LANES, 8, 128), jnp.float32),
            pltpu.VMEM((_LANES, 8, 128), jnp.float32),
            pltpu.VMEM((_LANES,), jnp.float32),
            pltpu.VMEM((_LANES,), jnp.float32),
            pltpu.SemaphoreType.DMA,
            pltpu.SemaphoreType.DMA,
        ],
    )


def _tc_soft_body(a_ref, g_ref, m_ref, out_ref):
    m = m_ref[...].astype(jnp.float32)
    cnt = jnp.sum(m, axis=1, keepdims=True)

    def masked_logsoftmax_sum(x):
        xmax = jnp.max(x, axis=1, keepdims=True)
        lse = jnp.log(jnp.sum(jnp.exp(x - xmax), axis=1, keepdims=True)) + xmax
        return jnp.sum(x * m) - jnp.sum(cnt * lse)

    out_ref[0] = masked_logsoftmax_sum(a_ref[...])
    out_ref[1] = masked_logsoftmax_sum(g_ref[...])
    out_ref[2] = jnp.sum(cnt)


def _tc_fin_body(s_ref, vp_ref, mp_ref, o0_ref, o1_ref, o2_ref):
    n_m = s_ref[2]
    o0_ref[0] = -jnp.sum(vp_ref[...]) / jnp.sum(mp_ref[...])
    o1_ref[0] = -s_ref[0] / n_m
    o2_ref[0] = -s_ref[1] / n_m


def _tc_soft(a, g, m):
    return pl.pallas_call(
        _tc_soft_body,
        out_shape=jax.ShapeDtypeStruct((3,), jnp.float32),
        out_specs=pl.BlockSpec(memory_space=pltpu.SMEM),
    )(a, g, m)


def _tc_fin(s, vp, mp):
    return pl.pallas_call(
        _tc_fin_body,
        in_specs=[pl.BlockSpec(memory_space=pltpu.SMEM),
                  pl.BlockSpec(memory_space=pltpu.VMEM),
                  pl.BlockSpec(memory_space=pltpu.VMEM)],
        out_shape=[jax.ShapeDtypeStruct((1,), jnp.float32)] * 3,
        out_specs=[pl.BlockSpec(memory_space=pltpu.SMEM)] * 3,
    )(s, vp, mp)


def kernel(txt_input, att2_weights, ground_weights, target, att2_target,
           input_seq):
    b, s = target.shape
    n = b * s
    tgt_p = target.astype(jnp.int32).reshape(n)[: _NACT * _BPW].reshape(
        _NACT, _BPW)
    vp, mp = _sc_gather()(txt_input.T, tgt_p)
    at = jnp.transpose(att2_weights, (1, 2, 0))
    gt = jnp.transpose(ground_weights, (1, 2, 0))
    mt = jnp.transpose(att2_target, (1, 2, 0))
    sums = _tc_soft(at, gt, mt)
    o0, o1, o2 = _tc_fin(sums, vp, mp)
    return o0[0], o1[0], o2[0]


# speedup vs baseline: 1.0671x; 1.0671x over previous
import functools

import jax
import jax.numpy as jnp
from jax import lax
from jax.experimental import pallas as pl
from jax.experimental.pallas import tpu as pltpu
from jax.experimental.pallas import tpu_sc as plsc

_NC = 2
_NS = 16
_NW = _NC * _NS
_LANES = 16
_NACT = 25
_BPW = 256
_GROUPS = _BPW // _LANES
_SUB = 8
_SEQLEN = 50


def _sc_gather_body(txt_hbm, tgt_hbm, outv_hbm, outm_hbm,
                    tgt_v, prev_v, ext_v, buf_a, buf_b, vred_v, mred_v,
                    sem_a, sem_b):
    wid = lax.axis_index("s") * _NC + lax.axis_index("c")
    lane = lax.iota(jnp.int32, _LANES)

    @pl.when(wid < _NACT)
    def _active():
        pltpu.sync_copy(tgt_hbm.at[wid], tgt_v)
        pltpu.sync_copy(
            tgt_hbm.at[jnp.maximum(wid - 1, 0), pl.ds(_BPW - _LANES, _LANES)],
            prev_v)
        ext_v[pl.ds(0, _LANES)] = prev_v[...]
        for g in range(_GROUPS):
            ext_v[pl.ds(_LANES + g * _LANES, _LANES)] = (
                tgt_v[pl.ds(g * _LANES, _LANES)])
        row0 = wid * _BPW
        bufs = (buf_a, buf_b)
        sems = (sem_a, sem_b)

        def group_vectors(g):
            tvec = tgt_v[pl.ds(g * _LANES, _LANES)]
            prev = ext_v[pl.ds(_LANES + g * _LANES - 1, _LANES)]
            kvec = row0 + g * _LANES + lane
            first = (kvec % _SEQLEN) == 0
            mvec = jnp.where(jnp.logical_or(first, prev > 0), 1.0, 0.0)
            return tvec, mvec

        def fire(g, parity, tvec):
            buf, sem = bufs[parity], sems[parity]
            cb = pl.multiple_of((row0 + g * _LANES) & ~127, 128)
            return [pltpu.async_copy(
                txt_hbm.at[pl.ds(pl.multiple_of(tvec[l] & ~7, 8), 8),
                           pl.ds(cb, 128)],
                buf.at[l], sem) for l in range(_LANES)]

        def outer(o, carry):
            accv, accm = carry
            g0 = o * _SUB
            gv = {0: group_vectors(g0)}
            cps = {0: fire(g0, 0, gv[0][0])}
            for gg in range(_SUB):
                if gg + 1 < _SUB:
                    gv[gg + 1] = group_vectors(g0 + gg + 1)
                    cps[gg + 1] = fire(g0 + gg + 1, (gg + 1) & 1,
                                       gv[gg + 1][0])
                tvec, mvec = gv[gg]
                buf = bufs[gg & 1]
                t7 = tvec & 7
                ws = [jnp.where(t7 == s, mvec, 0.0) for s in range(8)]
                for l in range(_LANES):
                    cps[gg][l].wait()
                for l in range(_LANES):
                    for s in range(8):
                        chunk = buf[l, s, pl.ds(gg * _LANES, _LANES)]
                        accv = accv + chunk * jnp.where(lane == l, ws[s], 0.0)
                accm = accm + mvec
            return accv, accm

        accv, accm = lax.fori_loop(
            0, _GROUPS // _SUB, outer,
            (jnp.zeros((_LANES,), jnp.float32),
             jnp.zeros((_LANES,), jnp.float32)))
        vred_v[...] = accv
        mred_v[...] = accm
        pltpu.sync_copy(vred_v, outv_hbm.at[wid])
        pltpu.sync_copy(mred_v, outm_hbm.at[wid])


@functools.lru_cache(maxsize=1)
def _sc_gather():
    return pl.kernel(
        _sc_gather_body,
        out_type=[jax.ShapeDtypeStruct((_NACT, _LANES), jnp.float32),
                  jax.ShapeDtypeStruct((_NACT, _LANES), jnp.float32)],
        mesh=plsc.VectorSubcoreMesh(core_axis_name="c", subcore_axis_name="s",
                                    num_cores=_NC),
        scratch_types=[
            pltpu.VMEM((_BPW,), jnp.int32),
            pltpu.VMEM((_LANES,), jnp.int32),
            pltpu.VMEM((_LANES + _BPW,), jnp.int32),
            pltpu.VMEM((_LANES, 8, 128), jnp.float32),
            pltpu.VMEM((_LANES, 8, 128), jnp.float32),
            pltpu.VMEM((_LANES,), jnp.float32),
            pltpu.VMEM((_LANES,), jnp.float32),
            pltpu.SemaphoreType.DMA,
            pltpu.SemaphoreType.DMA,
        ],
    )


def _tc_soft_body(a_ref, g_ref, m_ref, out_ref):
    m = m_ref[...].astype(jnp.float32)
    cnt = jnp.sum(m, axis=1, keepdims=True)

    def masked_logsoftmax_sum(x):
        xmax = jnp.max(x, axis=1, keepdims=True)
        lse = jnp.log(jnp.sum(jnp.exp(x - xmax), axis=1, keepdims=True)) + xmax
        return jnp.sum(x * m) - jnp.sum(cnt * lse)

    out_ref[0] = masked_logsoftmax_sum(a_ref[...])
    out_ref[1] = masked_logsoftmax_sum(g_ref[...])
    out_ref[2] = jnp.sum(cnt)


def _tc_fin_body(s_ref, vp_ref, mp_ref, o0_ref, o1_ref, o2_ref):
    n_m = s_ref[2]
    o0_ref[0] = -jnp.sum(vp_ref[...]) / jnp.sum(mp_ref[...])
    o1_ref[0] = -s_ref[0] / n_m
    o2_ref[0] = -s_ref[1] / n_m


def _tc_soft(a, g, m):
    return pl.pallas_call(
        _tc_soft_body,
        out_shape=jax.ShapeDtypeStruct((3,), jnp.float32),
        out_specs=pl.BlockSpec(memory_space=pltpu.SMEM),
    )(a, g, m)


def _tc_fin(s, vp, mp):
    return pl.pallas_call(
        _tc_fin_body,
        in_specs=[pl.BlockSpec(memory_space=pltpu.SMEM),
                  pl.BlockSpec(memory_space=pltpu.VMEM),
                  pl.BlockSpec(memory_space=pltpu.VMEM)],
        out_shape=[jax.ShapeDtypeStruct((1,), jnp.float32)] * 3,
        out_specs=[pl.BlockSpec(memory_space=pltpu.SMEM)] * 3,
    )(s, vp, mp)


def kernel(txt_input, att2_weights, ground_weights, target, att2_target,
           input_seq):
    b, s = target.shape
    n = b * s
    tgt_p = target.astype(jnp.int32).reshape(n)[: _NACT * _BPW].reshape(
        _NACT, _BPW)
    vp, mp = _sc_gather()(txt_input.T, tgt_p)
    at = jnp.transpose(att2_weights, (1, 2, 0))
    gt = jnp.transpose(ground_weights, (1, 2, 0))
    mt = jnp.transpose(att2_target, (1, 2, 0))
    sums = _tc_soft(at, gt, mt)
    o0, o1, o2 = _tc_fin(sums, vp, mp)
    return o0[0], o1[0], o2[0]


# speedup vs baseline: 1.0691x; 1.0019x over previous
import functools

import jax
import jax.numpy as jnp
from jax import lax
from jax.experimental import pallas as pl
from jax.experimental.pallas import tpu as pltpu
from jax.experimental.pallas import tpu_sc as plsc

_NC = 2
_NS = 16
_NW = _NC * _NS
_LANES = 16
_NACT = 25
_BPW = 256
_GROUPS = _BPW // _LANES
_SUB = 8
_SEQLEN = 50


def _sc_gather_body(txt_hbm, tgt_hbm, outv_hbm, outm_hbm,
                    tgt_v, prev_v, ext_v, buf_a, buf_b, vred_v, mred_v,
                    sem_a, sem_b):
    wid = lax.axis_index("s") * _NC + lax.axis_index("c")
    lane = lax.iota(jnp.int32, _LANES)

    @pl.when(wid < _NACT)
    def _active():
        pltpu.sync_copy(tgt_hbm.at[wid], tgt_v)
        pltpu.sync_copy(
            tgt_hbm.at[jnp.maximum(wid - 1, 0), pl.ds(_BPW - _LANES, _LANES)],
            prev_v)
        ext_v[pl.ds(0, _LANES)] = prev_v[...]
        for g in range(_GROUPS):
            ext_v[pl.ds(_LANES + g * _LANES, _LANES)] = (
                tgt_v[pl.ds(g * _LANES, _LANES)])
        row0 = wid * _BPW
        bufs = (buf_a, buf_b)
        sems = (sem_a, sem_b)

        def group_vectors(g):
            tvec = tgt_v[pl.ds(g * _LANES, _LANES)]
            prev = ext_v[pl.ds(_LANES + g * _LANES - 1, _LANES)]
            kvec = row0 + g * _LANES + lane
            first = (kvec % _SEQLEN) == 0
            mvec = jnp.where(jnp.logical_or(first, prev > 0), 1.0, 0.0)
            return tvec, mvec

        def fire(g, parity, tvec):
            buf, sem = bufs[parity], sems[parity]
            cb = pl.multiple_of((row0 + g * _LANES) & ~127, 128)
            return [pltpu.async_copy(
                txt_hbm.at[pl.ds(pl.multiple_of(tvec[l] & ~7, 8), 8),
                           pl.ds(cb, 128)],
                buf.at[l], sem) for l in range(_LANES)]

        def outer(o, carry):
            accv, accm = carry
            g0 = o * _SUB
            gv = {0: group_vectors(g0)}
            cps = {0: fire(g0, 0, gv[0][0])}
            for gg in range(_SUB):
                if gg + 1 < _SUB:
                    gv[gg + 1] = group_vectors(g0 + gg + 1)
                    cps[gg + 1] = fire(g0 + gg + 1, (gg + 1) & 1,
                                       gv[gg + 1][0])
                tvec, mvec = gv[gg]
                buf = bufs[gg & 1]
                t7 = tvec & 7
                ws = [jnp.where(t7 == s, mvec, 0.0) for s in range(8)]
                for l in range(_LANES):
                    cps[gg][l].wait()
                for l in range(_LANES):
                    for s in range(8):
                        chunk = buf[l, s, pl.ds(gg * _LANES, _LANES)]
                        accv = accv + chunk * jnp.where(lane == l, ws[s], 0.0)
                accm = accm + mvec
            return accv, accm

        accv, accm = lax.fori_loop(
            0, _GROUPS // _SUB, outer,
            (jnp.zeros((_LANES,), jnp.float32),
             jnp.zeros((_LANES,), jnp.float32)))
        vred_v[...] = accv
        mred_v[...] = accm
        pltpu.sync_copy(vred_v, outv_hbm.at[wid])
        pltpu.sync_copy(mred_v, outm_hbm.at[wid])


@functools.lru_cache(maxsize=1)
def _sc_gather():
    return pl.kernel(
        _sc_gather_body,
        out_type=[jax.ShapeDtypeStruct((_NACT, _LANES), jnp.float32),
                  jax.ShapeDtypeStruct((_NACT, _LANES), jnp.float32)],
        mesh=plsc.VectorSubcoreMesh(core_axis_name="c", subcore_axis_name="s",
                                    num_cores=_NC),
        scratch_types=[
            pltpu.VMEM((_BPW,), jnp.int32),
            pltpu.VMEM((_LANES,), jnp.int32),
            pltpu.VMEM((_LANES + _BPW,), jnp.int32),
            pltpu.VMEM((_LANES, 8, 128), jnp.float32),
            pltpu.VMEM((_LANES, 8, 128), jnp.float32),
            pltpu.VMEM((_LANES,), jnp.float32),
            pltpu.VMEM((_LANES,), jnp.float32),
            pltpu.SemaphoreType.DMA,
            pltpu.SemaphoreType.DMA,
        ],
    )


def _tc_soft_body(a_ref, g_ref, m_ref, o1_ref, o2_ref):
    m = m_ref[...].astype(jnp.float32)
    cnt = jnp.sum(m, axis=1, keepdims=True)
    n_m = jnp.sum(cnt)

    def masked_logsoftmax_sum(x):
        xmax = jnp.max(x, axis=1, keepdims=True)
        lse = jnp.log(jnp.sum(jnp.exp(x - xmax), axis=1, keepdims=True)) + xmax
        return jnp.sum(x * m) - jnp.sum(cnt * lse)

    o1_ref[0] = -masked_logsoftmax_sum(a_ref[...]) / n_m
    o2_ref[0] = -masked_logsoftmax_sum(g_ref[...]) / n_m


def _tc_fin_body(vp_ref, mp_ref, o0_ref):
    o0_ref[0] = -jnp.sum(vp_ref[...]) / jnp.sum(mp_ref[...])


def _tc_soft(a, g, m):
    return pl.pallas_call(
        _tc_soft_body,
        out_shape=[jax.ShapeDtypeStruct((1,), jnp.float32)] * 2,
        out_specs=[pl.BlockSpec(memory_space=pltpu.SMEM)] * 2,
    )(a, g, m)


def _tc_fin(vp, mp):
    return pl.pallas_call(
        _tc_fin_body,
        out_shape=[jax.ShapeDtypeStruct((1,), jnp.float32)],
        out_specs=[pl.BlockSpec(memory_space=pltpu.SMEM)],
    )(vp, mp)


def kernel(txt_input, att2_weights, ground_weights, target, att2_target,
           input_seq):
    b, s = target.shape
    n = b * s
    tgt_p = target.astype(jnp.int32).reshape(n)[: _NACT * _BPW].reshape(
        _NACT, _BPW)
    vp, mp = _sc_gather()(txt_input.T, tgt_p)
    at = jnp.transpose(att2_weights, (1, 2, 0))
    gt = jnp.transpose(ground_weights, (1, 2, 0))
    mt = jnp.transpose(att2_target, (1, 2, 0))
    o1, o2 = _tc_soft(at, gt, mt)
    (o0,) = _tc_fin(vp, mp)
    return o0[0], o1[0], o2[0]
